# baseline (speedup 1.0000x reference)
import jax
import jax.numpy as jnp
from jax.experimental import pallas as pl
from jax.experimental.pallas import tpu as pltpu
from jax.experimental.pallas import tpu_sc as plsc

ALPHA, LIMIT = 1.702, 7.0


def _sc_routing_body(idx_hbm, out_hbm, idx_v, ord_s, ord_v, sem):
    c = jax.lax.axis_index("c")
    s = jax.lax.axis_index("s")
    n_idx = idx_v.shape[0]
    n_exp = ord_v.shape[0] - 16

    @pl.when(jnp.logical_and(c == 0, s == 0))
    def _work():
        cp = pltpu.make_async_copy(idx_hbm, idx_v, sem)
        cp.start()
        cp.wait()
        for e in range(n_exp):
            ord_s[e] = 0
        for b in range(n_idx // 16):
            vb = idx_v[pl.ds(b * 16, 16)]
            for l in range(16):
                ord_s[vb[l]] = 1
        na = jnp.int32(0)
        for e in range(n_exp):
            a = ord_s[e]
            ord_s[na] = e
            na = na + a
        ord_s[n_exp] = na
        lane = jax.lax.iota(jnp.int32, 16)
        for ch in range(ord_v.shape[0] // 16):
            acc = jnp.zeros((16,), jnp.int32)
            for l in range(16):
                acc = jnp.where(lane == l, ord_s[ch * 16 + l], acc)
            ord_v[pl.ds(ch * 16, 16)] = acc
        cp2 = pltpu.make_async_copy(ord_v, out_hbm, sem)
        cp2.start()
        cp2.wait()


def _sc_routing(idx_flat, n_experts):
    n_pad = n_experts + 16
    return pl.kernel(
        _sc_routing_body,
        out_type=jax.ShapeDtypeStruct((n_pad,), jnp.int32),
        mesh=plsc.VectorSubcoreMesh(core_axis_name="c", subcore_axis_name="s"),
        scratch_types=[
            pltpu.VMEM((idx_flat.shape[0],), jnp.int32),
            pltpu.SMEM((n_pad,), jnp.int32),
            pltpu.VMEM((n_pad,), jnp.int32),
            pltpu.SemaphoreType.DMA,
        ],
        name="moe_routing_sc",
    )(idx_flat)


def _moe_body(sched_ref, x_ref, w1_hbm, b1_ref, w2_hbm, b2_ref, idx_ref,
              wgt_ref, out_ref,
              w1_buf, w2_buf, sems):
    i = pl.program_id(0)
    H = x_ref.shape[1]
    num_active = sched_ref[2 * pl.num_programs(0)]
    NBUF = 4

    def start_fetch(step, slot):
        for j in range(2):
            gid = 2 * step + j

            @pl.when(gid < num_active)
            def _s(j=j, gid=gid):
                e = sched_ref[gid]
                pltpu.make_async_copy(w1_hbm.at[e], w1_buf.at[slot, j], sems.at[slot, 2 * j]).start()
                pltpu.make_async_copy(w2_hbm.at[e], w2_buf.at[slot, j], sems.at[slot, 2 * j + 1]).start()

    def wait_fetch(slot, j):
        pltpu.make_async_copy(w1_hbm.at[0], w1_buf.at[slot, j], sems.at[slot, 2 * j]).wait()
        pltpu.make_async_copy(w2_hbm.at[0], w2_buf.at[slot, j], sems.at[slot, 2 * j + 1]).wait()

    @pl.when(i == 0)
    def _init():
        out_ref[...] = jnp.zeros_like(out_ref)
        for k in range(NBUF - 1):
            start_fetch(k, k)

    start_fetch(i + NBUF - 1, jax.lax.rem(i + NBUF - 1, NBUF))

    slot = jax.lax.rem(i, NBUF)
    for j in range(2):

        @pl.when(2 * i + j < num_active)
        def _compute(j=j):
            wait_fetch(slot, j)
            e = sched_ref[2 * i + j]

            idx = idx_ref[0]
            wgt = wgt_ref[0].astype(jnp.float32)
            c = jnp.sum(jnp.where(idx == e, wgt, 0.0), axis=1, keepdims=True)

            x = x_ref[...]
            x_glu = jax.lax.dot_general(
                x, w1_buf[slot, j, :, :H], (((1,), (1,)), ((), ())),
                preferred_element_type=jnp.float32)
            x_lin = jax.lax.dot_general(
                x, w1_buf[slot, j, :, H:], (((1,), (1,)), ((), ())),
                preferred_element_type=jnp.float32)
            b1 = b1_ref[e].astype(jnp.float32)
            x_glu = x_glu + b1[0:1, :]
            x_lin = x_lin + b1[1:2, :]
            x_glu = x_glu.astype(jnp.bfloat16).astype(jnp.float32)
            x_lin = x_lin.astype(jnp.bfloat16).astype(jnp.float32)

            x_glu = jnp.minimum(x_glu, LIMIT)
            x_lin = jnp.clip(x_lin, -LIMIT, LIMIT)
            act = (x_glu * jax.nn.sigmoid(ALPHA * x_glu)) * (x_lin + 1.0)
            act = act.astype(jnp.bfloat16)

            t2 = jax.lax.dot_general(
                act, w2_buf[slot, j], (((1,), (1,)), ((), ())),
                preferred_element_type=jnp.float32)
            t2 = t2 + b2_ref[e].astype(jnp.float32)

            out_ref[...] += t2 * c


def kernel(x, expert_weights, mlp1_weight, mlp1_bias, mlp2_weight, mlp2_bias,
           expert_indices):
    T, H = x.shape
    E, two_i, _ = mlp1_weight.shape
    K = expert_indices.shape[1]
    I = two_i // 2

    w1v = mlp1_weight.reshape(E, I, 2 * H)
    b1v = mlp1_bias.reshape(E, I, 2).transpose(0, 2, 1)
    b2r = mlp2_bias[:, None, :]
    idx32 = expert_indices.astype(jnp.int32)
    idx3 = idx32[None]
    wgt3 = expert_weights[None]

    sched = _sc_routing(idx32.reshape(-1), E)

    grid_spec = pltpu.PrefetchScalarGridSpec(
        num_scalar_prefetch=1,
        grid=(E // 2,),
        in_specs=[
            pl.BlockSpec((T, H), lambda i, s: (0, 0)),
            pl.BlockSpec(memory_space=pltpu.MemorySpace.HBM),
            pl.BlockSpec((E, 2, I), lambda i, s: (0, 0, 0)),
            pl.BlockSpec(memory_space=pltpu.MemorySpace.HBM),
            pl.BlockSpec((E, 1, H), lambda i, s: (0, 0, 0)),
            pl.BlockSpec((1, T, K), lambda i, s: (0, 0, 0)),
            pl.BlockSpec((1, T, K), lambda i, s: (0, 0, 0)),
        ],
        out_specs=pl.BlockSpec((T, H), lambda i, s: (0, 0)),
        scratch_shapes=[
            pltpu.VMEM((4, 2, I, 2 * H), jnp.bfloat16),
            pltpu.VMEM((4, 2, H, I), jnp.bfloat16),
            pltpu.SemaphoreType.DMA((4, 4)),
        ],
    )

    out = pl.pallas_call(
        _moe_body,
        grid_spec=grid_spec,
        out_shape=jax.ShapeDtypeStruct((T, H), jnp.float32),
        compiler_params=pltpu.CompilerParams(
            dimension_semantics=("arbitrary",)),
    )(sched, x, w1v, b1v, mlp2_weight, b2r, idx3, wgt3)

    return out.astype(x.dtype)


# speedup vs baseline: 1.0023x; 1.0023x over previous
import jax
import jax.numpy as jnp
from jax.experimental import pallas as pl
from jax.experimental.pallas import tpu as pltpu
from jax.experimental.pallas import tpu_sc as plsc

ALPHA, LIMIT = 1.702, 7.0


def _sc_routing_body(idx_hbm, out_hbm, idx_v, ord_s, ord_v, sem):
    c = jax.lax.axis_index("c")
    s = jax.lax.axis_index("s")
    n_idx = idx_v.shape[0]
    n_exp = ord_v.shape[0] - 16

    @pl.when(jnp.logical_and(c == 0, s == 0))
    def _work():
        cp = pltpu.make_async_copy(idx_hbm, idx_v, sem)
        cp.start()
        cp.wait()
        for e in range(n_exp):
            ord_s[e] = 0
        for b in range(n_idx // 16):
            vb = idx_v[pl.ds(b * 16, 16)]
            for l in range(16):
                ord_s[vb[l]] = 1
        na = jnp.int32(0)
        for e in range(n_exp):
            a = ord_s[e]
            ord_s[na] = e
            na = na + a
        ord_s[n_exp] = na
        lane = jax.lax.iota(jnp.int32, 16)
        for ch in range(ord_v.shape[0] // 16):
            acc = jnp.zeros((16,), jnp.int32)
            for l in range(16):
                acc = jnp.where(lane == l, ord_s[ch * 16 + l], acc)
            ord_v[pl.ds(ch * 16, 16)] = acc
        cp2 = pltpu.make_async_copy(ord_v, out_hbm, sem)
        cp2.start()
        cp2.wait()


def _sc_routing(idx_flat, n_experts):
    n_pad = n_experts + 16
    return pl.kernel(
        _sc_routing_body,
        out_type=jax.ShapeDtypeStruct((n_pad,), jnp.int32),
        mesh=plsc.VectorSubcoreMesh(core_axis_name="c", subcore_axis_name="s"),
        scratch_types=[
            pltpu.VMEM((idx_flat.shape[0],), jnp.int32),
            pltpu.SMEM((n_pad,), jnp.int32),
            pltpu.VMEM((n_pad,), jnp.int32),
            pltpu.SemaphoreType.DMA,
        ],
        name="moe_routing_sc",
    )(idx_flat)


def _moe_body(sched_ref, x_ref, w1_hbm, b1_ref, w2_hbm, b2_ref, idx_ref,
              wgt_ref, out_ref,
              w1_buf, w2_buf, sems):
    i = pl.program_id(0)
    H = x_ref.shape[1]
    num_active = sched_ref[2 * pl.num_programs(0)]
    NBUF = 4

    def start_fetch(step, slot):
        for j in range(2):
            gid = 2 * step + j

            @pl.when(gid < num_active)
            def _s(j=j, gid=gid):
                e = sched_ref[gid]
                pltpu.make_async_copy(w1_hbm.at[e], w1_buf.at[slot, j, pl.ds(0, w1_hbm.shape[1])], sems.at[slot, 2 * j]).start()
                pltpu.make_async_copy(w2_hbm.at[e], w2_buf.at[slot, j, pl.ds(0, w2_hbm.shape[1])], sems.at[slot, 2 * j + 1]).start()

    def wait_fetch(slot, j):
        pltpu.make_async_copy(w1_hbm.at[0], w1_buf.at[slot, j, pl.ds(0, w1_hbm.shape[1])], sems.at[slot, 2 * j]).wait()
        pltpu.make_async_copy(w2_hbm.at[0], w2_buf.at[slot, j, pl.ds(0, w2_hbm.shape[1])], sems.at[slot, 2 * j + 1]).wait()

    @pl.when(i == 0)
    def _init():
        out_ref[...] = jnp.zeros_like(out_ref)
        for k in range(NBUF - 1):
            start_fetch(k, k)

    start_fetch(i + NBUF - 1, jax.lax.rem(i + NBUF - 1, NBUF))

    slot = jax.lax.rem(i, NBUF)
    for j in range(2):

        @pl.when(2 * i + j < num_active)
        def _compute(j=j):
            wait_fetch(slot, j)
            e = sched_ref[2 * i + j]

            idx = idx_ref[0]
            wgt = wgt_ref[0].astype(jnp.float32)
            c = jnp.sum(jnp.where(idx == e, wgt, 0.0), axis=1, keepdims=True)

            x = x_ref[...]
            x_glu = jax.lax.dot_general(
                x, w1_buf[slot, j, :w1_hbm.shape[1], :H], (((1,), (1,)), ((), ())),
                preferred_element_type=jnp.float32)
            x_lin = jax.lax.dot_general(
                x, w1_buf[slot, j, :w1_hbm.shape[1], H:], (((1,), (1,)), ((), ())),
                preferred_element_type=jnp.float32)
            b1 = b1_ref[e].astype(jnp.float32)
            x_glu = x_glu + b1[0:1, :]
            x_lin = x_lin + b1[1:2, :]
            x_glu = x_glu.astype(jnp.bfloat16).astype(jnp.float32)
            x_lin = x_lin.astype(jnp.bfloat16).astype(jnp.float32)

            x_glu = jnp.minimum(x_glu, LIMIT)
            x_lin = jnp.clip(x_lin, -LIMIT, LIMIT)
            act = (x_glu * jax.nn.sigmoid(ALPHA * x_glu)) * (x_lin + 1.0)
            act = act.astype(jnp.bfloat16)

            t2 = jax.lax.dot_general(
                act, w2_buf[slot, j, :w2_hbm.shape[1]], (((1,), (1,)), ((), ())),
                preferred_element_type=jnp.float32)
            t2 = t2 + b2_ref[e].astype(jnp.float32)

            out_ref[...] += t2 * c


def kernel(x, expert_weights, mlp1_weight, mlp1_bias, mlp2_weight, mlp2_bias,
           expert_indices):
    T, H = x.shape
    E, two_i, _ = mlp1_weight.shape
    K = expert_indices.shape[1]
    I = two_i // 2

    w1v = mlp1_weight.reshape(E, I, 2 * H)
    b1v = mlp1_bias.reshape(E, I, 2).transpose(0, 2, 1)
    b2r = mlp2_bias[:, None, :]
    idx32 = expert_indices.astype(jnp.int32)
    idx3 = idx32[None]
    wgt3 = expert_weights[None]

    sched = _sc_routing(idx32.reshape(-1), E)

    grid_spec = pltpu.PrefetchScalarGridSpec(
        num_scalar_prefetch=1,
        grid=(E // 2,),
        in_specs=[
            pl.BlockSpec((T, H), lambda i, s: (0, 0)),
            pl.BlockSpec(memory_space=pltpu.MemorySpace.HBM),
            pl.BlockSpec((E, 2, I), lambda i, s: (0, 0, 0)),
            pl.BlockSpec(memory_space=pltpu.MemorySpace.HBM),
            pl.BlockSpec((E, 1, H), lambda i, s: (0, 0, 0)),
            pl.BlockSpec((1, T, K), lambda i, s: (0, 0, 0)),
            pl.BlockSpec((1, T, K), lambda i, s: (0, 0, 0)),
        ],
        out_specs=pl.BlockSpec((T, H), lambda i, s: (0, 0)),
        scratch_shapes=[
            pltpu.VMEM((4, 2, I + 8, 2 * H), jnp.bfloat16),
            pltpu.VMEM((4, 2, H + 8, I), jnp.bfloat16),
            pltpu.SemaphoreType.DMA((4, 4)),
        ],
    )

    out = pl.pallas_call(
        _moe_body,
        grid_spec=grid_spec,
        out_shape=jax.ShapeDtypeStruct((T, H), jnp.float32),
        compiler_params=pltpu.CompilerParams(
            dimension_semantics=("arbitrary",)),
    )(sched, x, w1v, b1v, mlp2_weight, b2r, idx3, wgt3)

    return out.astype(x.dtype)
